# baseline (speedup 1.0000x reference)
import functools

import jax
import jax.numpy as jnp
from jax import lax
from jax.experimental import pallas as pl
from jax.experimental.pallas import tpu as pltpu
from jax.experimental.pallas import tpu_sc as plsc

_D = 64
_B1 = 4096
_B2 = 200
_NW = 32
_NITER = 7
_KB = 4
_NBATCH = 8


@functools.partial(
    pl.kernel,
    mesh=plsc.VectorSubcoreMesh(core_axis_name="c", subcore_axis_name="s"),
    out_type=jax.ShapeDtypeStruct((_B2, 8, 32, 8, 128), jnp.float32),
    scratch_types=[
        pltpu.VMEM((_KB, 8, 128), jnp.int32),
        pltpu.VMEM((_KB * 128, _D), jnp.float32),
        pltpu.VMEM((2, 8, _KB, 8, 128), jnp.float32),
        pltpu.SemaphoreType.DMA,
        pltpu.SemaphoreType.DMA,
        pltpu.SemaphoreType.DMA,
    ],
    compiler_params=pltpu.CompilerParams(
        use_tc_tiling_on_sc=False, needs_layout_passes=False),
)
def _embed_fused(idx5_hbm, tab_hbm, out_hbm, slab_v, gbuf_v, obuf_v,
                 sem_s, sem_g, sem_w):
    w = lax.axis_index("s") * 2 + lax.axis_index("c")
    row_ids = [lax.iota(jnp.int32, 16) + c * 16 for c in range(8)]

    def body(i, carry):
        b2 = w + _NW * i

        @pl.when(b2 < _B2)
        def _():
            b2t = b2 // 8
            b2w = lax.rem(b2, 8)

            def load_slab(batch):
                return pltpu.async_copy(
                    idx5_hbm.at[b2t].at[pl.ds(batch * _KB, _KB)],
                    slab_v, sem_s)

            def fire(kk):
                return pltpu.async_copy(
                    tab_hbm.at[slab_v.at[kk].at[b2w]],
                    gbuf_v.at[pl.ds(kk * 128, 128)], sem_g)

            def write_half(batch, buf, start):
                maker = pltpu.async_copy if start else pltpu.make_async_copy
                return [
                    maker(
                        obuf_v.at[buf].at[jt],
                        out_hbm.at[b2].at[jt].at[pl.ds(batch * _KB, _KB)],
                        sem_w)
                    for jt in range(8)
                ]

            for batch in range(_NBATCH):
                buf = batch % 2
                load_slab(batch).wait()
                copies = [fire(kk) for kk in range(_KB)]
                if batch >= 2:
                    for cw in write_half(batch - 2, buf, False):
                        cw.wait()
                for kk in range(_KB):
                    copies[kk].wait()

                def tbody(t, buf=buf):
                    kk = t // 64
                    j = lax.rem(t, 64)
                    jt = j // 8
                    jw = lax.rem(j, 8)
                    base = kk * 128
                    col = jnp.full((16,), j, jnp.int32)
                    for c in range(8):
                        v = plsc.load_gather(
                            gbuf_v, [row_ids[c] + base, col])
                        obuf_v.at[buf].at[jt].at[kk].at[jw][
                            pl.ds(c * 16, 16)] = v

                plsc.parallel_loop(0, 64 * _KB, 1, unroll=8)(tbody)
                write_half(batch, buf, True)
            for batch in (_NBATCH - 2, _NBATCH - 1):
                for cw in write_half(batch, batch % 2, False):
                    cw.wait()

        return carry

    lax.fori_loop(0, _NITER, body, 0)


def kernel(input_sentence, table):
    idx5 = (
        jnp.transpose(input_sentence)
        .reshape(25, 8, 32, 128)
        .transpose(0, 2, 1, 3)
        .astype(jnp.int32)
    )
    out5d = _embed_fused(idx5, table)
    return out5d.transpose(2, 4, 0, 1, 3).reshape(_B1, _B2, _D)


# speedup vs baseline: 1.2152x; 1.2152x over previous
import functools

import jax
import jax.numpy as jnp
from jax import lax
from jax.experimental import pallas as pl
from jax.experimental.pallas import tpu as pltpu
from jax.experimental.pallas import tpu_sc as plsc

_D = 64
_B = 4096 * 200
_NW = 32
_BLK = 128
_NBLK = _B // _BLK
_BLK_PER_W = _NBLK // _NW
_K = 5
_NITER = _BLK_PER_W // _K
_NITER2 = _NITER // 2


@functools.partial(
    pl.kernel,
    mesh=plsc.VectorSubcoreMesh(core_axis_name="c", subcore_axis_name="s"),
    out_type=jax.ShapeDtypeStruct((_B, _D), jnp.float32),
    scratch_types=[
        pltpu.VMEM((2, _K, _BLK), jnp.int32),
        pltpu.VMEM((2, _K * _BLK, _D), jnp.float32),
        pltpu.SemaphoreType.DMA,
        pltpu.SemaphoreType.DMA,
        pltpu.SemaphoreType.DMA,
        pltpu.SemaphoreType.DMA,
        pltpu.SemaphoreType.DMA,
    ],
    compiler_params=pltpu.CompilerParams(use_tc_tiling_on_sc=False),
)
def _embed_gather(idx_hbm, table_hbm, out_hbm, idx_v, rows_v, sem_i0, sem_i1,
                  sem_w0, sem_w1, sem_g):
    wid = lax.axis_index("s") * 2 + lax.axis_index("c")
    base = wid * _BLK_PER_W
    rbase = wid * _BLK_PER_W * _BLK
    _CR = _K * _BLK
    sem_i = (sem_i0, sem_i1)
    sem_w = (sem_w0, sem_w1)

    def start_idx(chunk, buf):
        return pltpu.async_copy(
            idx_hbm.at[pl.ds(base + chunk * _K, _K)], idx_v.at[buf], sem_i[buf])

    def fire_gathers(buf):
        return [
            pltpu.async_copy(
                table_hbm.at[idx_v.at[buf].at[j]],
                rows_v.at[buf].at[pl.ds(j * _BLK, _BLK)], sem_g)
            for j in range(_K)
        ]

    def start_write(chunk, buf):
        return pltpu.async_copy(
            rows_v.at[buf], out_hbm.at[pl.ds(rbase + chunk * _CR, _CR)],
            sem_w[buf])

    start_idx(0, 0).wait()
    g = fire_gathers(0)
    start_idx(1, 1)
    for c in g:
        c.wait()
    start_write(0, 0)

    pltpu.make_async_copy(
        idx_hbm.at[pl.ds(base, _K)], idx_v.at[1], sem_i[1]).wait()
    g = fire_gathers(1)
    start_idx(2, 0)
    for c in g:
        c.wait()
    start_write(1, 1)

    def body(gi, carry):
        for b in (0, 1):
            chunk = 2 * gi + b
            row = base + chunk * _K
            pltpu.make_async_copy(
                idx_hbm.at[pl.ds(row, _K)], idx_v.at[b], sem_i[b]).wait()
            pltpu.make_async_copy(
                rows_v.at[b], out_hbm.at[pl.ds(rbase, _CR)], sem_w[b]).wait()
            g = fire_gathers(b)
            nxt = lax.rem(chunk + 1, _NITER)
            start_idx(nxt, 1 - b)
            for c in g:
                c.wait()
            start_write(chunk, b)
        return carry

    lax.fori_loop(1, _NITER2, body, 0)

    pltpu.make_async_copy(
        idx_hbm.at[pl.ds(base, _K)], idx_v.at[0], sem_i[0]).wait()
    pltpu.make_async_copy(
        rows_v.at[0], out_hbm.at[pl.ds(rbase, _CR)], sem_w[0]).wait()
    pltpu.make_async_copy(
        rows_v.at[1], out_hbm.at[pl.ds(rbase, _CR)], sem_w[1]).wait()


def kernel(input_sentence, table):
    idx = input_sentence.reshape(_NBLK, _BLK).astype(jnp.int32)
    out = _embed_gather(idx, table)
    return out.reshape(input_sentence.shape + (_D,))
